# baseline (speedup 1.0000x reference)
import functools

import jax
import jax.numpy as jnp
from jax import lax
from jax.experimental import pallas as pl
from jax.experimental.pallas import tpu as pltpu
from jax.experimental.pallas import tpu_sc as plsc

_CODEBOOK = 1024
_DIM = 64
_ROWS = 16384
_R = 512
_G = _ROWS // _R
_HALVES = 1024 // _R
_BETA = 0.25

_NC = 2
_NS = 16
_NW = _NC * _NS
_BPW = _ROWS // _NW
_L = 16


def _argmin_body(xt_ref, emb_ref, idx_ref, loss_ref):
    xt = xt_ref[...].reshape(_DIM, _R)
    sx = jnp.sum(xt * xt, axis=0, keepdims=True)
    tt = lax.dot_general(emb_ref[...], xt, (((1,), (0,)), ((), ())),
                         preferred_element_type=jnp.float32)
    se = jnp.sum(emb_ref[...] ** 2, axis=1, keepdims=True)
    d = (sx - 2.0 * tt) + se
    m = jnp.min(d, axis=0, keepdims=True)
    codes = lax.broadcasted_iota(jnp.int32, d.shape, 0)
    idx = jnp.min(jnp.where(d == m, codes, _CODEBOOK), axis=0,
                  keepdims=True)
    idx_ref[...] = idx.reshape(1, 1, _R)
    loss_ref[...] = jnp.sum(m).reshape(1, 1, 1)


def _tc_argmin(xt3, e):
    return pl.pallas_call(
        _argmin_body,
        grid=(_G,),
        in_specs=[
            pl.BlockSpec((1, _DIM, _R),
                         lambda i: (i // _HALVES, 0, i % _HALVES)),
            pl.BlockSpec((_CODEBOOK, _DIM), lambda i: (0, 0)),
        ],
        out_specs=[
            pl.BlockSpec((1, 1, _R), lambda i: (i // _HALVES, 0, i % _HALVES)),
            pl.BlockSpec((1, 1, 1), lambda i: (i, 0, 0)),
        ],
        out_shape=[
            jax.ShapeDtypeStruct((16, 1, 1024), jnp.int32),
            jax.ShapeDtypeStruct((_G, 1, 1), jnp.float32),
        ],
        compiler_params=pltpu.CompilerParams(
            dimension_semantics=("parallel",)),
    )(xt3, e)


@functools.partial(
    pl.kernel,
    out_type=jax.ShapeDtypeStruct((16, _DIM, 1024), jnp.float32),
    mesh=plsc.VectorSubcoreMesh(core_axis_name="c", subcore_axis_name="s",
                                num_cores=_NC, num_subcores=_NS),
    scratch_types=[
        pltpu.VMEM((_BPW,), jnp.int32),
        pltpu.VMEM((_DIM, _CODEBOOK), jnp.float32),
        pltpu.VMEM((_DIM, _BPW), jnp.float32),
    ],
    compiler_params=pltpu.CompilerParams(use_tc_tiling_on_sc=True,
                                         needs_layout_passes=False),
)
def _sc_gather_t(embt_hbm, idx_hbm, out_hbm, idx_v, embt_v, out_v):
    wid = lax.axis_index("s") * _NC + lax.axis_index("c")
    b = wid // 2
    half = wid % 2
    pltpu.sync_copy(idx_hbm.at[pl.ds(wid * _BPW, _BPW)], idx_v)
    pltpu.sync_copy(embt_hbm, embt_v)

    @pl.loop(0, _BPW // _L)
    def _(k):
        idx16 = idx_v[pl.ds(k * _L, _L)]
        for dd in range(_DIM):
            row = jnp.full((_L,), dd, jnp.int32)
            out_v[dd, pl.ds(k * _L, _L)] = plsc.load_gather(
                embt_v, [row, idx16])

    pltpu.sync_copy(out_v, out_hbm.at[b, :, pl.ds(half * _BPW, _BPW)])


def kernel(z_e, emb):
    bsz, num_slots, code_dim = z_e.shape
    xt = lax.transpose(z_e.astype(jnp.float32), (0, 2, 1))
    e = emb.astype(jnp.float32)
    et = e.T

    idx3, loss_parts = _tc_argmin(xt, e)
    out_t = _sc_gather_t(et, idx3.reshape(_ROWS))
    loss = jnp.sum(loss_parts) * ((1.0 + _BETA) / float(_ROWS * _DIM))

    return (lax.transpose(out_t, (0, 2, 1)),
            idx3.reshape(bsz, num_slots),
            loss)


# speedup vs baseline: 1.0644x; 1.0644x over previous
import functools

import jax
import jax.numpy as jnp
from jax import lax
from jax.experimental import pallas as pl
from jax.experimental.pallas import tpu as pltpu
from jax.experimental.pallas import tpu_sc as plsc

_CODEBOOK = 1024
_DIM = 64
_SLOTS = 1024
_BATCH = 16
_CHUNKB = 8
_R = 512
_HALVES = _SLOTS // _R
_GC = _CHUNKB * _HALVES
_BETA = 0.25

_NC = 2
_NS = 16
_L = 16
_DG = 4
_DPW = _DIM // _DG


def _argmin_body(xt_ref, emb_ref, idx_ref, loss_ref):
    xt = xt_ref[...].reshape(_DIM, _R)
    sx = jnp.sum(xt * xt, axis=0, keepdims=True)
    tt = lax.dot_general(emb_ref[...], xt, (((1,), (0,)), ((), ())),
                         preferred_element_type=jnp.float32)
    se = jnp.sum(emb_ref[...] ** 2, axis=1, keepdims=True)
    d = (sx - 2.0 * tt) + se
    m = jnp.min(d, axis=0, keepdims=True)
    codes = lax.broadcasted_iota(jnp.int32, d.shape, 0)
    idx = jnp.min(jnp.where(d == m, codes, _CODEBOOK), axis=0,
                  keepdims=True)
    idx_ref[...] = idx.reshape(1, 1, _R)
    loss_ref[...] = jnp.sum(m).reshape(1, 1, 1)


def _tc_argmin(xt, e, b0):
    return pl.pallas_call(
        _argmin_body,
        grid=(_GC,),
        in_specs=[
            pl.BlockSpec((1, _DIM, _R),
                         lambda i: (b0 + i // _HALVES, 0, i % _HALVES)),
            pl.BlockSpec((_CODEBOOK, _DIM), lambda i: (0, 0)),
        ],
        out_specs=[
            pl.BlockSpec((1, 1, _R), lambda i: (i // _HALVES, 0, i % _HALVES)),
            pl.BlockSpec((1, 1, 1), lambda i: (i, 0, 0)),
        ],
        out_shape=[
            jax.ShapeDtypeStruct((_CHUNKB, 1, _SLOTS), jnp.int32),
            jax.ShapeDtypeStruct((_GC, 1, 1), jnp.float32),
        ],
        compiler_params=pltpu.CompilerParams(
            dimension_semantics=("parallel",)),
    )(xt, e)


@functools.partial(
    pl.kernel,
    out_type=jax.ShapeDtypeStruct((_CHUNKB, _DIM, _SLOTS), jnp.float32),
    mesh=plsc.VectorSubcoreMesh(core_axis_name="c", subcore_axis_name="s",
                                num_cores=_NC, num_subcores=_NS),
    scratch_types=[
        pltpu.VMEM((_SLOTS,), jnp.int32),
        pltpu.VMEM((_DPW, _CODEBOOK), jnp.float32),
        pltpu.VMEM((_DPW, _SLOTS), jnp.float32),
    ],
    compiler_params=pltpu.CompilerParams(use_tc_tiling_on_sc=True,
                                         needs_layout_passes=False),
)
def _sc_gather_t(embt_hbm, idx_hbm, out_hbm, idx_v, embt_v, out_v):
    wid = lax.axis_index("s") * _NC + lax.axis_index("c")
    b = wid // _DG
    dg = wid % _DG
    pltpu.sync_copy(idx_hbm.at[pl.ds(b * _SLOTS, _SLOTS)], idx_v)
    pltpu.sync_copy(embt_hbm.at[pl.ds(dg * _DPW, _DPW), :], embt_v)

    @pl.loop(0, _SLOTS // _L)
    def _(k):
        idx16 = idx_v[pl.ds(k * _L, _L)]
        for dd in range(_DPW):
            row = jnp.full((_L,), dd, jnp.int32)
            out_v[dd, pl.ds(k * _L, _L)] = plsc.load_gather(
                embt_v, [row, idx16])

    pltpu.sync_copy(out_v, out_hbm.at[b, pl.ds(dg * _DPW, _DPW), :])


def kernel(z_e, emb):
    bsz, num_slots, code_dim = z_e.shape
    xt = lax.transpose(z_e.astype(jnp.float32), (0, 2, 1))
    e = emb.astype(jnp.float32)
    et = e.T

    idx_a, loss_a = _tc_argmin(xt, e, 0)
    out_a = _sc_gather_t(et, idx_a.reshape(_CHUNKB * _SLOTS))
    idx_b, loss_b = _tc_argmin(xt, e, _CHUNKB)
    out_b = _sc_gather_t(et, idx_b.reshape(_CHUNKB * _SLOTS))

    idx3 = jnp.concatenate([idx_a, idx_b], axis=0)
    out_t = jnp.concatenate([out_a, out_b], axis=0)
    loss = ((jnp.sum(loss_a) + jnp.sum(loss_b))
            * ((1.0 + _BETA) / float(_BATCH * _SLOTS * _DIM)))

    return (lax.transpose(out_t, (0, 2, 1)),
            idx3.reshape(bsz, num_slots),
            loss)


# speedup vs baseline: 1.0969x; 1.0305x over previous
import functools

import jax
import jax.numpy as jnp
from jax import lax
from jax.experimental import pallas as pl
from jax.experimental.pallas import tpu as pltpu
from jax.experimental.pallas import tpu_sc as plsc

_CODEBOOK = 1024
_DIM = 64
_SLOTS = 1024
_BATCH = 16
_CHUNKB = 8
_R = 512
_HALVES = _SLOTS // _R
_GC = _CHUNKB * _HALVES
_BETA = 0.25

_NC = 2
_NS = 16
_L = 16
_DG = 4
_DPW = _DIM // _DG


def _argmin_body(xt_ref, emb_ref, idx_ref, loss_ref):
    xt = xt_ref[...].reshape(_DIM, _R)
    sx = jnp.sum(xt * xt, axis=0, keepdims=True)
    tt = lax.dot_general(emb_ref[...], xt, (((1,), (0,)), ((), ())),
                         preferred_element_type=jnp.float32)
    se = jnp.sum(emb_ref[...] ** 2, axis=1, keepdims=True)
    d = (sx - 2.0 * tt) + se
    m = jnp.min(d, axis=0, keepdims=True)
    codes = lax.broadcasted_iota(jnp.int32, d.shape, 0)
    idx = jnp.min(jnp.where(d == m, codes, _CODEBOOK), axis=0,
                  keepdims=True)
    idx_ref[...] = idx.reshape(1, 1, _R)
    loss_ref[...] = jnp.sum(m).reshape(1, 1, 1)


def _vq_body(xt_ref, emb_ref, embt_ref, idx_ref, out_ref, loss_ref):
    xt = xt_ref[...].reshape(_DIM, _R)
    sx = jnp.sum(xt * xt, axis=0, keepdims=True)
    tt = lax.dot_general(emb_ref[...], xt, (((1,), (0,)), ((), ())),
                         preferred_element_type=jnp.float32)
    se = jnp.sum(emb_ref[...] ** 2, axis=1, keepdims=True)
    d = (sx - 2.0 * tt) + se
    m = jnp.min(d, axis=0, keepdims=True)
    codes = lax.broadcasted_iota(jnp.int32, d.shape, 0)
    idx = jnp.min(jnp.where(d == m, codes, _CODEBOOK), axis=0,
                  keepdims=True)
    idx_ref[...] = idx.reshape(1, 1, _R)
    onehot = (codes == idx).astype(jnp.float32)
    zqt = lax.dot_general(embt_ref[...], onehot, (((1,), (0,)), ((), ())),
                          preferred_element_type=jnp.float32)
    out_ref[...] = (xt + (zqt - xt)).reshape(1, _DIM, _R)
    loss_ref[...] = jnp.sum(m).reshape(1, 1, 1)


def _tc_vq(xt, e, et, b0):
    return pl.pallas_call(
        _vq_body,
        grid=(_GC,),
        in_specs=[
            pl.BlockSpec((1, _DIM, _R),
                         lambda i: (b0 + i // _HALVES, 0, i % _HALVES)),
            pl.BlockSpec((_CODEBOOK, _DIM), lambda i: (0, 0)),
            pl.BlockSpec((_DIM, _CODEBOOK), lambda i: (0, 0)),
        ],
        out_specs=[
            pl.BlockSpec((1, 1, _R), lambda i: (i // _HALVES, 0, i % _HALVES)),
            pl.BlockSpec((1, _DIM, _R),
                         lambda i: (i // _HALVES, 0, i % _HALVES)),
            pl.BlockSpec((1, 1, 1), lambda i: (i, 0, 0)),
        ],
        out_shape=[
            jax.ShapeDtypeStruct((_CHUNKB, 1, _SLOTS), jnp.int32),
            jax.ShapeDtypeStruct((_CHUNKB, _DIM, _SLOTS), jnp.float32),
            jax.ShapeDtypeStruct((_GC, 1, 1), jnp.float32),
        ],
        compiler_params=pltpu.CompilerParams(
            dimension_semantics=("parallel",)),
    )(xt, e, et)


def _tc_argmin(xt, e, b0):
    return pl.pallas_call(
        _argmin_body,
        grid=(_GC,),
        in_specs=[
            pl.BlockSpec((1, _DIM, _R),
                         lambda i: (b0 + i // _HALVES, 0, i % _HALVES)),
            pl.BlockSpec((_CODEBOOK, _DIM), lambda i: (0, 0)),
        ],
        out_specs=[
            pl.BlockSpec((1, 1, _R), lambda i: (i // _HALVES, 0, i % _HALVES)),
            pl.BlockSpec((1, 1, 1), lambda i: (i, 0, 0)),
        ],
        out_shape=[
            jax.ShapeDtypeStruct((_CHUNKB, 1, _SLOTS), jnp.int32),
            jax.ShapeDtypeStruct((_GC, 1, 1), jnp.float32),
        ],
        compiler_params=pltpu.CompilerParams(
            dimension_semantics=("parallel",)),
    )(xt, e)


@functools.partial(
    pl.kernel,
    out_type=jax.ShapeDtypeStruct((_CHUNKB, _DIM, _SLOTS), jnp.float32),
    mesh=plsc.VectorSubcoreMesh(core_axis_name="c", subcore_axis_name="s",
                                num_cores=_NC, num_subcores=_NS),
    scratch_types=[
        pltpu.VMEM((_SLOTS,), jnp.int32),
        pltpu.VMEM((_DPW, _CODEBOOK), jnp.float32),
        pltpu.VMEM((_DPW, _SLOTS), jnp.float32),
    ],
    compiler_params=pltpu.CompilerParams(use_tc_tiling_on_sc=True,
                                         needs_layout_passes=False),
)
def _sc_gather_t(embt_hbm, idx_hbm, out_hbm, idx_v, embt_v, out_v):
    wid = lax.axis_index("s") * _NC + lax.axis_index("c")
    b = wid // _DG
    dg = wid % _DG
    pltpu.sync_copy(idx_hbm.at[pl.ds(b * _SLOTS, _SLOTS)], idx_v)
    pltpu.sync_copy(embt_hbm.at[pl.ds(dg * _DPW, _DPW), :], embt_v)

    @pl.loop(0, _SLOTS // _L)
    def _(k):
        idx16 = idx_v[pl.ds(k * _L, _L)]
        for dd in range(_DPW):
            row = jnp.full((_L,), dd, jnp.int32)
            out_v[dd, pl.ds(k * _L, _L)] = plsc.load_gather(
                embt_v, [row, idx16])

    pltpu.sync_copy(out_v, out_hbm.at[b, pl.ds(dg * _DPW, _DPW), :])


def kernel(z_e, emb):
    bsz, num_slots, code_dim = z_e.shape
    xt = lax.transpose(z_e.astype(jnp.float32), (0, 2, 1))
    e = emb.astype(jnp.float32)
    et = e.T

    idx_a, loss_a = _tc_argmin(xt, e, 0)
    out_a = _sc_gather_t(et, idx_a.reshape(_CHUNKB * _SLOTS))
    idx_b, out_b, loss_b = _tc_vq(xt, e, et, _CHUNKB)

    idx3 = jnp.concatenate([idx_a, idx_b], axis=0)
    out_t = jnp.concatenate([out_a, out_b], axis=0)
    loss = ((jnp.sum(loss_a) + jnp.sum(loss_b))
            * ((1.0 + _BETA) / float(_BATCH * _SLOTS * _DIM)))

    return (lax.transpose(out_t, (0, 2, 1)),
            idx3.reshape(bsz, num_slots),
            loss)


# speedup vs baseline: 1.2542x; 1.1434x over previous
import functools

import jax
import jax.numpy as jnp
from jax import lax
from jax.experimental import pallas as pl
from jax.experimental.pallas import tpu as pltpu
from jax.experimental.pallas import tpu_sc as plsc

_CODEBOOK = 1024
_DIM = 64
_SLOTS = 1024
_BATCH = 16
_CHUNKB = 8
_R = 1024
_HALVES = _SLOTS // _R
_GC = _CHUNKB * _HALVES
_BETA = 0.25

_NC = 2
_NS = 16
_L = 16
_DG = 4
_DPW = _DIM // _DG


def _argmin_body(xt_ref, emb_ref, idx_ref, loss_ref):
    xt = xt_ref[...].reshape(_DIM, _R)
    sx = jnp.sum(xt * xt, axis=0, keepdims=True)
    tt = lax.dot_general(emb_ref[...], xt, (((1,), (0,)), ((), ())),
                         preferred_element_type=jnp.float32)
    se = jnp.sum(emb_ref[...] ** 2, axis=1, keepdims=True)
    d = (sx - 2.0 * tt) + se
    m = jnp.min(d, axis=0, keepdims=True)
    codes = lax.broadcasted_iota(jnp.int32, d.shape, 0)
    idx = jnp.min(jnp.where(d == m, codes, _CODEBOOK), axis=0,
                  keepdims=True)
    idx_ref[...] = idx.reshape(1, 1, _R)
    loss_ref[...] = jnp.sum(m).reshape(1, 1, 1)


def _vq_body(xt_ref, emb_ref, embt_ref, idx_ref, out_ref, loss_ref):
    xt = xt_ref[...].reshape(_DIM, _R)
    sx = jnp.sum(xt * xt, axis=0, keepdims=True)
    tt = lax.dot_general(emb_ref[...], xt, (((1,), (0,)), ((), ())),
                         preferred_element_type=jnp.float32)
    se = jnp.sum(emb_ref[...] ** 2, axis=1, keepdims=True)
    d = (sx - 2.0 * tt) + se
    m = jnp.min(d, axis=0, keepdims=True)
    codes = lax.broadcasted_iota(jnp.int32, d.shape, 0)
    idx = jnp.min(jnp.where(d == m, codes, _CODEBOOK), axis=0,
                  keepdims=True)
    idx_ref[...] = idx.reshape(1, 1, _R)
    onehot = (codes == idx).astype(jnp.float32)
    zqt = lax.dot_general(embt_ref[...], onehot, (((1,), (0,)), ((), ())),
                          preferred_element_type=jnp.float32)
    out_ref[...] = (xt + (zqt - xt)).reshape(1, _DIM, _R)
    loss_ref[...] = jnp.sum(m).reshape(1, 1, 1)


def _tc_vq(xt, e, et, b0):
    return pl.pallas_call(
        _vq_body,
        grid=(_GC,),
        in_specs=[
            pl.BlockSpec((1, _DIM, _R),
                         lambda i: (b0 + i // _HALVES, 0, i % _HALVES)),
            pl.BlockSpec((_CODEBOOK, _DIM), lambda i: (0, 0)),
            pl.BlockSpec((_DIM, _CODEBOOK), lambda i: (0, 0)),
        ],
        out_specs=[
            pl.BlockSpec((1, 1, _R), lambda i: (i // _HALVES, 0, i % _HALVES)),
            pl.BlockSpec((1, _DIM, _R),
                         lambda i: (i // _HALVES, 0, i % _HALVES)),
            pl.BlockSpec((1, 1, 1), lambda i: (i, 0, 0)),
        ],
        out_shape=[
            jax.ShapeDtypeStruct((_CHUNKB, 1, _SLOTS), jnp.int32),
            jax.ShapeDtypeStruct((_CHUNKB, _DIM, _SLOTS), jnp.float32),
            jax.ShapeDtypeStruct((_GC, 1, 1), jnp.float32),
        ],
        compiler_params=pltpu.CompilerParams(
            dimension_semantics=("parallel",)),
    )(xt, e, et)


def _tc_argmin(xt, e, b0):
    return pl.pallas_call(
        _argmin_body,
        grid=(_GC,),
        in_specs=[
            pl.BlockSpec((1, _DIM, _R),
                         lambda i: (b0 + i // _HALVES, 0, i % _HALVES)),
            pl.BlockSpec((_CODEBOOK, _DIM), lambda i: (0, 0)),
        ],
        out_specs=[
            pl.BlockSpec((1, 1, _R), lambda i: (i // _HALVES, 0, i % _HALVES)),
            pl.BlockSpec((1, 1, 1), lambda i: (i, 0, 0)),
        ],
        out_shape=[
            jax.ShapeDtypeStruct((_CHUNKB, 1, _SLOTS), jnp.int32),
            jax.ShapeDtypeStruct((_GC, 1, 1), jnp.float32),
        ],
        compiler_params=pltpu.CompilerParams(
            dimension_semantics=("parallel",)),
    )(xt, e)


@functools.partial(
    pl.kernel,
    out_type=jax.ShapeDtypeStruct((_CHUNKB, _DIM, _SLOTS), jnp.float32),
    mesh=plsc.VectorSubcoreMesh(core_axis_name="c", subcore_axis_name="s",
                                num_cores=_NC, num_subcores=_NS),
    scratch_types=[
        pltpu.VMEM((_SLOTS,), jnp.int32),
        pltpu.VMEM((_DPW, _CODEBOOK), jnp.float32),
        pltpu.VMEM((_DPW, _SLOTS), jnp.float32),
    ],
    compiler_params=pltpu.CompilerParams(use_tc_tiling_on_sc=True,
                                         needs_layout_passes=False),
)
def _sc_gather_t(embt_hbm, idx_hbm, out_hbm, idx_v, embt_v, out_v):
    wid = lax.axis_index("s") * _NC + lax.axis_index("c")
    b = wid // _DG
    dg = wid % _DG
    pltpu.sync_copy(idx_hbm.at[pl.ds(b * _SLOTS, _SLOTS)], idx_v)
    pltpu.sync_copy(embt_hbm.at[pl.ds(dg * _DPW, _DPW), :], embt_v)

    @pl.loop(0, _SLOTS // _L)
    def _(k):
        idx16 = idx_v[pl.ds(k * _L, _L)]
        for dd in range(_DPW):
            row = jnp.full((_L,), dd, jnp.int32)
            out_v[dd, pl.ds(k * _L, _L)] = plsc.load_gather(
                embt_v, [row, idx16])

    pltpu.sync_copy(out_v, out_hbm.at[b, pl.ds(dg * _DPW, _DPW), :])


def kernel(z_e, emb):
    bsz, num_slots, code_dim = z_e.shape
    xt = lax.transpose(z_e.astype(jnp.float32), (0, 2, 1))
    e = emb.astype(jnp.float32)
    et = e.T

    idx_a, loss_a = _tc_argmin(xt, e, 0)
    out_a = _sc_gather_t(et, idx_a.reshape(_CHUNKB * _SLOTS))
    idx_b, out_b, loss_b = _tc_vq(xt, e, et, _CHUNKB)

    idx3 = jnp.concatenate([idx_a, idx_b], axis=0)
    out_t = jnp.concatenate([out_a, out_b], axis=0)
    loss = ((jnp.sum(loss_a) + jnp.sum(loss_b))
            * ((1.0 + _BETA) / float(_BATCH * _SLOTS * _DIM)))

    return (lax.transpose(out_t, (0, 2, 1)),
            idx3.reshape(bsz, num_slots),
            loss)
